# baseline (speedup 1.0000x reference)
import functools
import jax
import jax.numpy as jnp
from jax import lax
from jax.experimental import pallas as pl
from jax.experimental.pallas import tpu as pltpu
from jax.experimental.pallas import tpu_sc as plsc

_OFFSET = 2
_MAX_POS = 2048
_D_MODEL = 1024
_NC = 2
_NS = 16
_NW = _NC * _NS
_RPW = _MAX_POS // _NW

_mesh = plsc.VectorSubcoreMesh(core_axis_name="c", subcore_axis_name="s")


@functools.partial(
    pl.kernel,
    out_type=jax.ShapeDtypeStruct((_MAX_POS, _D_MODEL), jnp.float32),
    mesh=_mesh,
    scratch_types=[
        pltpu.VMEM((_RPW + 8, _D_MODEL), jnp.float32),
        pltpu.SemaphoreType.DMA,
        pltpu.SemaphoreType.DMA,
    ],
    compiler_params=pltpu.CompilerParams(use_tc_tiling_on_sc=False),
)
def _sc_slice_copy(table_hbm, out_hbm, buf_v, sem0, sem1):
    wid = lax.axis_index("s") * _NC + lax.axis_index("c")
    base = pl.multiple_of(wid * _RPW, 8)
    main = pltpu.async_copy(
        table_hbm.at[pl.ds(base, _RPW), :], buf_v.at[pl.ds(0, _RPW), :], sem0
    )

    @pl.when(wid < _NW - 1)
    def _tail_full():
        pltpu.async_copy(
            table_hbm.at[pl.ds(base + _RPW, 8), :],
            buf_v.at[pl.ds(_RPW, 8), :],
            sem1,
        ).wait()

    @pl.when(wid == _NW - 1)
    def _tail_edge():
        pltpu.async_copy(
            table_hbm.at[pl.ds(base + _RPW, _OFFSET), :],
            buf_v.at[pl.ds(_RPW, _OFFSET), :],
            sem1,
        ).wait()

    main.wait()
    pltpu.sync_copy(
        buf_v.at[pl.ds(_OFFSET, _RPW), :], out_hbm.at[pl.ds(base, _RPW), :]
    )


def kernel(position_weights):
    return _sc_slice_copy(position_weights)


# speedup vs baseline: 5.6245x; 5.6245x over previous
import jax
import jax.numpy as jnp
from jax.experimental import pallas as pl
from jax.experimental.pallas import tpu as pltpu

_OFFSET = 2
_MAX_POS = 2048
_D_MODEL = 1024
_B = 512
_G = _MAX_POS // _B


def _shift_copy_kernel(big_ref, carry_ref, out_ref):
    out_ref[0 : _B - _OFFSET, :] = big_ref[_OFFSET:_B, :]
    out_ref[_B - _OFFSET : _B, :] = carry_ref[0:_OFFSET, :]


def kernel(position_weights):
    return pl.pallas_call(
        _shift_copy_kernel,
        grid=(_G,),
        in_specs=[
            pl.BlockSpec((_B, _D_MODEL), lambda i: (i, 0)),
            pl.BlockSpec((8, _D_MODEL), lambda i: ((i + 1) * (_B // 8), 0)),
        ],
        out_specs=pl.BlockSpec((_B, _D_MODEL), lambda i: (i, 0)),
        out_shape=jax.ShapeDtypeStruct((_MAX_POS, _D_MODEL), jnp.float32),
        compiler_params=pltpu.CompilerParams(
            dimension_semantics=("arbitrary",),
        ),
    )(position_weights, position_weights)


# speedup vs baseline: 6.9376x; 1.2335x over previous
import jax
import jax.numpy as jnp
from jax.experimental import pallas as pl
from jax.experimental.pallas import tpu as pltpu

_OFFSET = 2
_MAX_POS = 2048
_D_MODEL = 1024
_B = 1024
_G = _MAX_POS // _B


def _shift_copy_kernel(big_ref, carry_ref, out_ref):
    out_ref[0 : _B - _OFFSET, :] = big_ref[_OFFSET:_B, :]
    out_ref[_B - _OFFSET : _B, :] = carry_ref[0:_OFFSET, :]


def kernel(position_weights):
    return pl.pallas_call(
        _shift_copy_kernel,
        grid=(_G,),
        in_specs=[
            pl.BlockSpec((_B, _D_MODEL), lambda i: (i, 0)),
            pl.BlockSpec((8, _D_MODEL), lambda i: ((i + 1) * (_B // 8), 0)),
        ],
        out_specs=pl.BlockSpec((_B, _D_MODEL), lambda i: (i, 0)),
        out_shape=jax.ShapeDtypeStruct((_MAX_POS, _D_MODEL), jnp.float32),
        compiler_params=pltpu.CompilerParams(
            dimension_semantics=("arbitrary",),
        ),
    )(position_weights, position_weights)


# speedup vs baseline: 7.0503x; 1.0162x over previous
import jax
import jax.numpy as jnp
from jax.experimental import pallas as pl
from jax.experimental.pallas import tpu as pltpu

_OFFSET = 2
_MAX_POS = 2048
_D_MODEL = 1024
_CH = 4
_CR = _MAX_POS // _CH


def _shift_copy_kernel(in_hbm, out_hbm, bufs, obufs, insems, tailsem, outsems):
    reads = []
    for c in range(_CH):
        rows = _CR + 8 if c < _CH - 1 else _CR
        reads.append(
            pltpu.make_async_copy(
                in_hbm.at[pl.ds(c * _CR, rows), :],
                bufs.at[c, pl.ds(0, rows), :],
                insems.at[c],
            )
        )
    tail = pltpu.make_async_copy(
        in_hbm.at[pl.ds(_MAX_POS, _OFFSET), :],
        bufs.at[_CH - 1, pl.ds(_CR, _OFFSET), :],
        tailsem,
    )
    for r in reads:
        r.start()
    tail.start()

    writes = []
    for c in range(_CH):
        reads[c].wait()
        if c == _CH - 1:
            tail.wait()
        obufs[c, :, :] = bufs[c, pl.ds(_OFFSET, _CR), :]
        w = pltpu.make_async_copy(
            obufs.at[c], out_hbm.at[pl.ds(c * _CR, _CR), :], outsems.at[c]
        )
        w.start()
        writes.append(w)
    for w in writes:
        w.wait()


def kernel(position_weights):
    return pl.pallas_call(
        _shift_copy_kernel,
        in_specs=[pl.BlockSpec(memory_space=pl.ANY)],
        out_specs=pl.BlockSpec(memory_space=pl.ANY),
        scratch_shapes=[
            pltpu.VMEM((_CH, _CR + 8, _D_MODEL), jnp.float32),
            pltpu.VMEM((_CH, _CR, _D_MODEL), jnp.float32),
            pltpu.SemaphoreType.DMA((_CH,)),
            pltpu.SemaphoreType.DMA,
            pltpu.SemaphoreType.DMA((_CH,)),
        ],
        out_shape=jax.ShapeDtypeStruct((_MAX_POS, _D_MODEL), jnp.float32),
    )(position_weights)
